# baseline (device time: 48529 ns/iter reference)
import jax
import jax.numpy as jnp
from jax import lax
from jax.experimental import pallas as pl
from jax.experimental.pallas import tpu as pltpu

N_DEV = 4
B_LOC = 2
SQ = 256
SKV = 256
HQ = 16
HQ_LOC = 4
DH = 64
D_MODEL = 512
HD_LOC = HQ_LOC * DH
WINDOW = 128


def _body(x_ref, wq_ref, k_hbm, v_hbm, wo_ref, out_ref,
          xb_ref, wqb_ref, wob_ref, kf_ref, vf_ref, kb_ref, vb_ref,
          cwq, cwo, ctx_ref, local_sems, swq, rwq, swo, rwo):
    my = lax.axis_index("i")

    local_cps = []
    for b in range(B_LOC):
        for hh in range(HQ):
            cp_k = pltpu.make_async_copy(
                k_hbm.at[my * B_LOC + b, :, hh, :],
                kf_ref.at[b * HQ + hh], local_sems.at[0])
            cp_v = pltpu.make_async_copy(
                v_hbm.at[my * B_LOC + b, :, hh, :],
                vf_ref.at[b * HQ + hh], local_sems.at[1])
            cp_k.start()
            cp_v.start()
            local_cps.append((cp_k, cp_v))

    wqb_ref[...] = wq_ref[...].astype(jnp.bfloat16)
    wob_ref[...] = wo_ref[...].astype(jnp.bfloat16)

    bar = pltpu.get_barrier_semaphore()
    for r in range(1, N_DEV):
        peer = lax.rem(my + r, N_DEV)
        pl.semaphore_signal(bar, inc=1, device_id=(peer,),
                            device_id_type=pl.DeviceIdType.MESH)
    pl.semaphore_wait(bar, N_DEV - 1)

    sends = []
    for r in range(1, N_DEV):
        peer = lax.rem(my + r, N_DEV)
        rd_q = pltpu.make_async_remote_copy(
            src_ref=wqb_ref, dst_ref=cwq.at[r - 1],
            send_sem=swq.at[r - 1], recv_sem=rwq.at[r - 1],
            device_id=(peer,), device_id_type=pl.DeviceIdType.MESH)
        rd_o = pltpu.make_async_remote_copy(
            src_ref=wob_ref, dst_ref=cwo.at[r - 1],
            send_sem=swo.at[r - 1], recv_sem=rwo.at[r - 1],
            device_id=(peer,), device_id_type=pl.DeviceIdType.MESH)
        rd_q.start()
        rd_o.start()
        sends.append((rd_q, rd_o))

    for b in range(B_LOC):
        xb_ref[b * SQ:(b + 1) * SQ, :] = x_ref[b].astype(jnp.bfloat16)
    for cp_k, cp_v in local_cps:
        cp_k.wait()
        cp_v.wait()
    kb_ref[...] = kf_ref[...].astype(jnp.bfloat16)
    vb_ref[...] = vf_ref[...].astype(jnp.bfloat16)

    qi = lax.broadcasted_iota(jnp.int32, (SQ, SKV), 0)
    ki = lax.broadcasted_iota(jnp.int32, (SQ, SKV), 1)
    mask = jnp.abs(qi - ki) <= WINDOW

    def compute_chunk(origin, wq_c, wo_c, first):
        q = jnp.dot(xb_ref[...], wq_c, preferred_element_type=jnp.float32)
        q = (q * 0.125).astype(jnp.bfloat16)
        for b in range(B_LOC):
            kv_start = b * HQ + origin * HQ_LOC
            kb = kb_ref[pl.ds(kv_start, HQ_LOC)]
            vb = vb_ref[pl.ds(kv_start, HQ_LOC)]
            for h in range(HQ_LOC):
                qh = q[b * SQ:(b + 1) * SQ, h * DH:(h + 1) * DH]
                s = lax.dot_general(
                    qh, kb[h], (((1,), (1,)), ((), ())),
                    preferred_element_type=jnp.float32)
                s = jnp.where(mask, s, -1e9)
                s = s - jnp.max(s, axis=1, keepdims=True)
                w = jnp.exp(s)
                w = (w / jnp.sum(w, axis=1, keepdims=True)).astype(jnp.bfloat16)
                ctx = jnp.dot(w, vb[h], preferred_element_type=jnp.float32)
                ctx_ref[b * SQ:(b + 1) * SQ, h * DH:(h + 1) * DH] = (
                    ctx.astype(jnp.bfloat16))
        for b in range(B_LOC):
            part = jnp.dot(ctx_ref[b * SQ:(b + 1) * SQ, :], wo_c,
                           preferred_element_type=jnp.float32)
            if first:
                out_ref[b] = part
            else:
                out_ref[b] = out_ref[b] + part

    compute_chunk(my, wqb_ref[...], wob_ref[...], first=True)
    for r in (1, 3, 2):
        rd_q, rd_o = sends[r - 1]
        rd_q.wait_recv()
        rd_o.wait_recv()
        origin = lax.rem(my - r + N_DEV, N_DEV)
        compute_chunk(origin, cwq[r - 1], cwo[r - 1], first=False)

    for rd_q, rd_o in sends:
        rd_q.wait_send()
        rd_o.wait_send()


def kernel(x, Wq, K_ext, V_ext, Wo):
    return pl.pallas_call(
        _body,
        out_shape=jax.ShapeDtypeStruct((B_LOC, SQ, D_MODEL), jnp.float32),
        in_specs=[
            pl.BlockSpec(memory_space=pltpu.VMEM),
            pl.BlockSpec(memory_space=pltpu.VMEM),
            pl.BlockSpec(memory_space=pl.ANY),
            pl.BlockSpec(memory_space=pl.ANY),
            pl.BlockSpec(memory_space=pltpu.VMEM),
        ],
        out_specs=pl.BlockSpec(memory_space=pltpu.VMEM),
        scratch_shapes=[
            pltpu.VMEM((B_LOC * SQ, D_MODEL), jnp.bfloat16),
            pltpu.VMEM((D_MODEL, HD_LOC), jnp.bfloat16),
            pltpu.VMEM((HD_LOC, D_MODEL), jnp.bfloat16),
            pltpu.VMEM((B_LOC * HQ, SKV, DH), jnp.float32),
            pltpu.VMEM((B_LOC * HQ, SKV, DH), jnp.float32),
            pltpu.VMEM((B_LOC * HQ, SKV, DH), jnp.bfloat16),
            pltpu.VMEM((B_LOC * HQ, SKV, DH), jnp.bfloat16),
            pltpu.VMEM((N_DEV - 1, D_MODEL, HD_LOC), jnp.bfloat16),
            pltpu.VMEM((N_DEV - 1, HD_LOC, D_MODEL), jnp.bfloat16),
            pltpu.VMEM((B_LOC * SQ, HD_LOC), jnp.bfloat16),
            pltpu.SemaphoreType.DMA((2,)),
            pltpu.SemaphoreType.DMA((N_DEV - 1,)),
            pltpu.SemaphoreType.DMA((N_DEV - 1,)),
            pltpu.SemaphoreType.DMA((N_DEV - 1,)),
            pltpu.SemaphoreType.DMA((N_DEV - 1,)),
        ],
        compiler_params=pltpu.CompilerParams(collective_id=0),
    )(x, Wq, K_ext, V_ext, Wo)


# device time: 26225 ns/iter; 1.8505x vs baseline; 1.8505x over previous
import jax
import jax.numpy as jnp
from jax import lax
from jax.experimental import pallas as pl
from jax.experimental.pallas import tpu as pltpu

N_DEV = 4
B_LOC = 2
SQ = 256
SKV = 256
HQ = 16
HQ_LOC = 4
DH = 64
D_MODEL = 512
HD_LOC = HQ_LOC * DH
WINDOW = 128


def _body(x_ref, wq_ref, kb_ref, vb_ref, wo_ref, out_ref,
          xb_ref, wqb_ref, wob_ref, cwq, cwo, ctx_ref,
          swq, rwq, swo, rwo):
    my = lax.axis_index("i")

    wqb_ref[...] = wq_ref[...].astype(jnp.bfloat16)
    wob_ref[...] = wo_ref[...].astype(jnp.bfloat16)

    bar = pltpu.get_barrier_semaphore()
    for r in range(1, N_DEV):
        peer = lax.rem(my + r, N_DEV)
        pl.semaphore_signal(bar, inc=1, device_id=(peer,),
                            device_id_type=pl.DeviceIdType.MESH)
    pl.semaphore_wait(bar, N_DEV - 1)

    sends = []
    for r in range(1, N_DEV):
        peer = lax.rem(my + r, N_DEV)
        rd_q = pltpu.make_async_remote_copy(
            src_ref=wqb_ref, dst_ref=cwq.at[r - 1],
            send_sem=swq.at[r - 1], recv_sem=rwq.at[r - 1],
            device_id=(peer,), device_id_type=pl.DeviceIdType.MESH)
        rd_o = pltpu.make_async_remote_copy(
            src_ref=wob_ref, dst_ref=cwo.at[r - 1],
            send_sem=swo.at[r - 1], recv_sem=rwo.at[r - 1],
            device_id=(peer,), device_id_type=pl.DeviceIdType.MESH)
        rd_q.start()
        rd_o.start()
        sends.append((rd_q, rd_o))

    for b in range(B_LOC):
        xb_ref[b * SQ:(b + 1) * SQ, :] = x_ref[b].astype(jnp.bfloat16)

    qi = lax.broadcasted_iota(jnp.int32, (SQ, SKV), 0)
    ki = lax.broadcasted_iota(jnp.int32, (SQ, SKV), 1)
    mask = jnp.abs(qi - ki) <= WINDOW

    def compute_chunk(origin, wq_c, wo_c, first):
        q = jnp.dot(xb_ref[...], wq_c, preferred_element_type=jnp.float32)
        q = (q * 0.125).astype(jnp.bfloat16)
        for b in range(B_LOC):
            kv_start = b * HQ + origin * HQ_LOC
            kb = kb_ref[pl.ds(kv_start, HQ_LOC)]
            vb = vb_ref[pl.ds(kv_start, HQ_LOC)]
            for h in range(HQ_LOC):
                qh = q[b * SQ:(b + 1) * SQ, h * DH:(h + 1) * DH]
                s = lax.dot_general(
                    qh, kb[h], (((1,), (1,)), ((), ())),
                    preferred_element_type=jnp.float32)
                s = jnp.where(mask, s, -1e9)
                s = s - jnp.max(s, axis=1, keepdims=True)
                w = jnp.exp(s)
                w = (w / jnp.sum(w, axis=1, keepdims=True)).astype(jnp.bfloat16)
                ctx = jnp.dot(w, vb[h], preferred_element_type=jnp.float32)
                ctx_ref[b * SQ:(b + 1) * SQ, h * DH:(h + 1) * DH] = (
                    ctx.astype(jnp.bfloat16))
        for b in range(B_LOC):
            part = jnp.dot(ctx_ref[b * SQ:(b + 1) * SQ, :], wo_c,
                           preferred_element_type=jnp.float32)
            if first:
                out_ref[b] = part
            else:
                out_ref[b] = out_ref[b] + part

    compute_chunk(my, wqb_ref[...], wob_ref[...], first=True)
    for r in (1, 3, 2):
        rd_q, rd_o = sends[r - 1]
        rd_q.wait_recv()
        rd_o.wait_recv()
        origin = lax.rem(my - r + N_DEV, N_DEV)
        compute_chunk(origin, cwq[r - 1], cwo[r - 1], first=False)

    for rd_q, rd_o in sends:
        rd_q.wait_send()
        rd_o.wait_send()


def kernel(x, Wq, K_ext, V_ext, Wo):
    my = lax.axis_index("i")
    Kb = lax.dynamic_slice_in_dim(K_ext, my * B_LOC, B_LOC, axis=0)
    Vb = lax.dynamic_slice_in_dim(V_ext, my * B_LOC, B_LOC, axis=0)
    Kb = Kb.astype(jnp.bfloat16).transpose(0, 2, 1, 3).reshape(
        B_LOC * HQ, SKV, DH)
    Vb = Vb.astype(jnp.bfloat16).transpose(0, 2, 1, 3).reshape(
        B_LOC * HQ, SKV, DH)

    return pl.pallas_call(
        _body,
        out_shape=jax.ShapeDtypeStruct((B_LOC, SQ, D_MODEL), jnp.float32),
        in_specs=[pl.BlockSpec(memory_space=pltpu.VMEM)] * 5,
        out_specs=pl.BlockSpec(memory_space=pltpu.VMEM),
        scratch_shapes=[
            pltpu.VMEM((B_LOC * SQ, D_MODEL), jnp.bfloat16),
            pltpu.VMEM((D_MODEL, HD_LOC), jnp.bfloat16),
            pltpu.VMEM((HD_LOC, D_MODEL), jnp.bfloat16),
            pltpu.VMEM((N_DEV - 1, D_MODEL, HD_LOC), jnp.bfloat16),
            pltpu.VMEM((N_DEV - 1, HD_LOC, D_MODEL), jnp.bfloat16),
            pltpu.VMEM((B_LOC * SQ, HD_LOC), jnp.bfloat16),
            pltpu.SemaphoreType.DMA((N_DEV - 1,)),
            pltpu.SemaphoreType.DMA((N_DEV - 1,)),
            pltpu.SemaphoreType.DMA((N_DEV - 1,)),
            pltpu.SemaphoreType.DMA((N_DEV - 1,)),
        ],
        compiler_params=pltpu.CompilerParams(collective_id=0),
    )(x, Wq, Kb, Vb, Wo)


# device time: 24577 ns/iter; 1.9746x vs baseline; 1.0671x over previous
import jax
import jax.numpy as jnp
from jax import lax
from jax.experimental import pallas as pl
from jax.experimental.pallas import tpu as pltpu

N_DEV = 4
B_LOC = 2
SQ = 256
SKV = 256
HQ = 16
HQ_LOC = 4
DH = 64
D_MODEL = 512
HD_LOC = HQ_LOC * DH
WINDOW = 128


def _body(x_ref, wq_ref, kb_ref, vb_ref, wo_ref, out_ref,
          xb_ref, wqb_ref, wob_ref, cwq, cwo, ctx_ref,
          swq, rwq, swo, rwo):
    my = lax.axis_index("i")

    wqb_ref[...] = wq_ref[...].astype(jnp.bfloat16)
    wob_ref[...] = wo_ref[...].astype(jnp.bfloat16)

    bar = pltpu.get_barrier_semaphore()
    for r in range(1, N_DEV):
        peer = lax.rem(my + r, N_DEV)
        pl.semaphore_signal(bar, inc=1, device_id=(peer,),
                            device_id_type=pl.DeviceIdType.MESH)
    pl.semaphore_wait(bar, N_DEV - 1)

    sends = []
    for r in range(1, N_DEV):
        peer = lax.rem(my + r, N_DEV)
        rd_q = pltpu.make_async_remote_copy(
            src_ref=wqb_ref, dst_ref=cwq.at[r - 1],
            send_sem=swq.at[r - 1], recv_sem=rwq.at[r - 1],
            device_id=(peer,), device_id_type=pl.DeviceIdType.MESH)
        rd_o = pltpu.make_async_remote_copy(
            src_ref=wob_ref, dst_ref=cwo.at[r - 1],
            send_sem=swo.at[r - 1], recv_sem=rwo.at[r - 1],
            device_id=(peer,), device_id_type=pl.DeviceIdType.MESH)
        rd_q.start()
        rd_o.start()
        sends.append((rd_q, rd_o))

    for b in range(B_LOC):
        xb_ref[b * SQ:(b + 1) * SQ, :] = x_ref[b].astype(jnp.bfloat16)

    qi = lax.broadcasted_iota(jnp.int32, (SQ, SKV), 0)
    ki = lax.broadcasted_iota(jnp.int32, (SQ, SKV), 1)
    mask = jnp.abs(qi - ki) <= WINDOW

    def compute_chunk(origin, wq_c, get_wo_c, first):
        q = jnp.dot(xb_ref[...], wq_c, preferred_element_type=jnp.float32)
        q = (q * 0.125).astype(jnp.bfloat16)
        for b in range(B_LOC):
            kv_start = b * HQ + origin * HQ_LOC
            kb = kb_ref[pl.ds(kv_start, HQ_LOC)]
            vb = vb_ref[pl.ds(kv_start, HQ_LOC)]
            for h in range(HQ_LOC):
                qh = q[b * SQ:(b + 1) * SQ, h * DH:(h + 1) * DH]
                s = lax.dot_general(
                    qh, kb[h], (((1,), (1,)), ((), ())),
                    preferred_element_type=jnp.float32)
                s = jnp.where(mask, s, -1e9)
                s = s - jnp.max(s, axis=1, keepdims=True)
                w = jnp.exp(s)
                w = (w / jnp.sum(w, axis=1, keepdims=True)).astype(jnp.bfloat16)
                ctx = jnp.dot(w, vb[h], preferred_element_type=jnp.float32)
                ctx_ref[b * SQ:(b + 1) * SQ, h * DH:(h + 1) * DH] = (
                    ctx.astype(jnp.bfloat16))
        wo_c = get_wo_c()
        for b in range(B_LOC):
            part = jnp.dot(ctx_ref[b * SQ:(b + 1) * SQ, :], wo_c,
                           preferred_element_type=jnp.float32)
            if first:
                out_ref[b] = part
            else:
                out_ref[b] = out_ref[b] + part

    compute_chunk(my, wqb_ref[...], lambda: wob_ref[...], first=True)
    for r in (1, 3, 2):
        rd_q, rd_o = sends[r - 1]
        rd_q.wait_recv()

        def get_wo(r=r, rd_o=rd_o):
            rd_o.wait_recv()
            return cwo[r - 1]

        origin = lax.rem(my - r + N_DEV, N_DEV)
        compute_chunk(origin, cwq[r - 1], get_wo, first=False)

    for rd_q, rd_o in sends:
        rd_q.wait_send()
        rd_o.wait_send()


def kernel(x, Wq, K_ext, V_ext, Wo):
    my = lax.axis_index("i")
    Kb = lax.dynamic_slice_in_dim(K_ext, my * B_LOC, B_LOC, axis=0)
    Vb = lax.dynamic_slice_in_dim(V_ext, my * B_LOC, B_LOC, axis=0)
    Kb = Kb.astype(jnp.bfloat16).transpose(0, 2, 1, 3).reshape(
        B_LOC * HQ, SKV, DH)
    Vb = Vb.astype(jnp.bfloat16).transpose(0, 2, 1, 3).reshape(
        B_LOC * HQ, SKV, DH)

    return pl.pallas_call(
        _body,
        out_shape=jax.ShapeDtypeStruct((B_LOC, SQ, D_MODEL), jnp.float32),
        in_specs=[pl.BlockSpec(memory_space=pltpu.VMEM)] * 5,
        out_specs=pl.BlockSpec(memory_space=pltpu.VMEM),
        scratch_shapes=[
            pltpu.VMEM((B_LOC * SQ, D_MODEL), jnp.bfloat16),
            pltpu.VMEM((D_MODEL, HD_LOC), jnp.bfloat16),
            pltpu.VMEM((HD_LOC, D_MODEL), jnp.bfloat16),
            pltpu.VMEM((N_DEV - 1, D_MODEL, HD_LOC), jnp.bfloat16),
            pltpu.VMEM((N_DEV - 1, HD_LOC, D_MODEL), jnp.bfloat16),
            pltpu.VMEM((B_LOC * SQ, HD_LOC), jnp.bfloat16),
            pltpu.SemaphoreType.DMA((N_DEV - 1,)),
            pltpu.SemaphoreType.DMA((N_DEV - 1,)),
            pltpu.SemaphoreType.DMA((N_DEV - 1,)),
            pltpu.SemaphoreType.DMA((N_DEV - 1,)),
        ],
        compiler_params=pltpu.CompilerParams(collective_id=0),
    )(x, Wq, Kb, Vb, Wo)


# device time: 24244 ns/iter; 2.0017x vs baseline; 1.0137x over previous
import jax
import jax.numpy as jnp
from jax import lax
from jax.experimental import pallas as pl
from jax.experimental.pallas import tpu as pltpu

N_DEV = 4
B_LOC = 2
SQ = 256
SKV = 256
HQ = 16
HQ_LOC = 4
DH = 64
D_MODEL = 512
HD_LOC = HQ_LOC * DH
WINDOW = 128


def _body(x_ref, wqi_ref, kb_ref, vb_ref, woi_ref, sc_ref, out_ref,
          xb_ref, cwq, cwo, csc, ctx_ref,
          swq, rwq, swo, rwo, ssc, rsc):
    my = lax.axis_index("i")

    bar = pltpu.get_barrier_semaphore()
    for r in range(1, N_DEV):
        peer = lax.rem(my + r, N_DEV)
        pl.semaphore_signal(bar, inc=1, device_id=(peer,),
                            device_id_type=pl.DeviceIdType.MESH)
    pl.semaphore_wait(bar, N_DEV - 1)

    sends = []
    for r in range(1, N_DEV):
        peer = lax.rem(my + r, N_DEV)
        rd_s = pltpu.make_async_remote_copy(
            src_ref=sc_ref, dst_ref=csc.at[r - 1],
            send_sem=ssc.at[r - 1], recv_sem=rsc.at[r - 1],
            device_id=(peer,), device_id_type=pl.DeviceIdType.MESH)
        rd_q = pltpu.make_async_remote_copy(
            src_ref=wqi_ref, dst_ref=cwq.at[r - 1],
            send_sem=swq.at[r - 1], recv_sem=rwq.at[r - 1],
            device_id=(peer,), device_id_type=pl.DeviceIdType.MESH)
        rd_o = pltpu.make_async_remote_copy(
            src_ref=woi_ref, dst_ref=cwo.at[r - 1],
            send_sem=swo.at[r - 1], recv_sem=rwo.at[r - 1],
            device_id=(peer,), device_id_type=pl.DeviceIdType.MESH)
        rd_s.start()
        rd_q.start()
        rd_o.start()
        sends.append((rd_s, rd_q, rd_o))

    for b in range(B_LOC):
        xb_ref[b * SQ:(b + 1) * SQ, :] = x_ref[b].astype(jnp.bfloat16)

    qi = lax.broadcasted_iota(jnp.int32, (SQ, SKV), 0)
    ki = lax.broadcasted_iota(jnp.int32, (SQ, SKV), 1)
    mask = jnp.abs(qi - ki) <= WINDOW

    def compute_chunk(origin, wq_i8, sq, so, get_wo_i8, first):
        wq_bf = wq_i8.astype(jnp.bfloat16)
        q = jnp.dot(xb_ref[...], wq_bf, preferred_element_type=jnp.float32)
        q = (q * (sq * 0.125)).astype(jnp.bfloat16)
        for b in range(B_LOC):
            kv_start = b * HQ + origin * HQ_LOC
            kb = kb_ref[pl.ds(kv_start, HQ_LOC)]
            vb = vb_ref[pl.ds(kv_start, HQ_LOC)]
            for h in range(HQ_LOC):
                qh = q[b * SQ:(b + 1) * SQ, h * DH:(h + 1) * DH]
                s = lax.dot_general(
                    qh, kb[h], (((1,), (1,)), ((), ())),
                    preferred_element_type=jnp.float32)
                s = jnp.where(mask, s, -1e9)
                s = s - jnp.max(s, axis=1, keepdims=True)
                w = jnp.exp(s)
                w = (w / jnp.sum(w, axis=1, keepdims=True)).astype(jnp.bfloat16)
                ctx = jnp.dot(w, vb[h], preferred_element_type=jnp.float32)
                ctx_ref[b * SQ:(b + 1) * SQ, h * DH:(h + 1) * DH] = (
                    ctx.astype(jnp.bfloat16))
        wo_bf = get_wo_i8().astype(jnp.bfloat16)
        for b in range(B_LOC):
            ctx_s = (ctx_ref[b * SQ:(b + 1) * SQ, :] * so).astype(jnp.bfloat16)
            part = jnp.dot(ctx_s, wo_bf, preferred_element_type=jnp.float32)
            if first:
                out_ref[b] = part
            else:
                out_ref[b] = out_ref[b] + part

    compute_chunk(my, wqi_ref[...], sc_ref[0:1, :], sc_ref[1:2, :],
                  lambda: woi_ref[...], first=True)
    for r in (1, 3, 2):
        rd_s, rd_q, rd_o = sends[r - 1]
        rd_s.wait_recv()
        rd_q.wait_recv()

        def get_wo(r=r, rd_o=rd_o):
            rd_o.wait_recv()
            return cwo[r - 1]

        origin = lax.rem(my - r + N_DEV, N_DEV)
        compute_chunk(origin, cwq[r - 1], csc[r - 1, 0:1, :],
                      csc[r - 1, 1:2, :], get_wo, first=False)

    for rd_s, rd_q, rd_o in sends:
        rd_s.wait_send()
        rd_q.wait_send()
        rd_o.wait_send()


def kernel(x, Wq, K_ext, V_ext, Wo):
    my = lax.axis_index("i")
    Kb = lax.dynamic_slice_in_dim(K_ext, my * B_LOC, B_LOC, axis=0)
    Vb = lax.dynamic_slice_in_dim(V_ext, my * B_LOC, B_LOC, axis=0)
    Kb = Kb.astype(jnp.bfloat16).transpose(0, 2, 1, 3).reshape(
        B_LOC * HQ, SKV, DH)
    Vb = Vb.astype(jnp.bfloat16).transpose(0, 2, 1, 3).reshape(
        B_LOC * HQ, SKV, DH)

    sq = jnp.maximum(jnp.max(jnp.abs(Wq), axis=0), 1e-6) / 127.0
    Wq_i8 = jnp.round(Wq / sq[None, :]).astype(jnp.int8)
    so = jnp.maximum(jnp.max(jnp.abs(Wo), axis=1), 1e-6) / 127.0
    Wo_i8 = jnp.round(Wo / so[:, None]).astype(jnp.int8)
    scales = jnp.stack([sq, so], axis=0)

    return pl.pallas_call(
        _body,
        out_shape=jax.ShapeDtypeStruct((B_LOC, SQ, D_MODEL), jnp.float32),
        in_specs=[pl.BlockSpec(memory_space=pltpu.VMEM)] * 6,
        out_specs=pl.BlockSpec(memory_space=pltpu.VMEM),
        scratch_shapes=[
            pltpu.VMEM((B_LOC * SQ, D_MODEL), jnp.bfloat16),
            pltpu.VMEM((N_DEV - 1, D_MODEL, HD_LOC), jnp.int8),
            pltpu.VMEM((N_DEV - 1, HD_LOC, D_MODEL), jnp.int8),
            pltpu.VMEM((N_DEV - 1, 2, HD_LOC), jnp.float32),
            pltpu.VMEM((B_LOC * SQ, HD_LOC), jnp.bfloat16),
            pltpu.SemaphoreType.DMA((N_DEV - 1,)),
            pltpu.SemaphoreType.DMA((N_DEV - 1,)),
            pltpu.SemaphoreType.DMA((N_DEV - 1,)),
            pltpu.SemaphoreType.DMA((N_DEV - 1,)),
            pltpu.SemaphoreType.DMA((N_DEV - 1,)),
            pltpu.SemaphoreType.DMA((N_DEV - 1,)),
        ],
        compiler_params=pltpu.CompilerParams(collective_id=0),
    )(x, Wq_i8, Kb, Vb, Wo_i8, scales)


# device time: 19840 ns/iter; 2.4460x vs baseline; 1.2220x over previous
import jax
import jax.numpy as jnp
from jax import lax
from jax.experimental import pallas as pl
from jax.experimental.pallas import tpu as pltpu

N_DEV = 4
B_LOC = 2
SQ = 256
SKV = 256
HQ = 16
HQ_LOC = 4
DH = 64
D_MODEL = 512
HD_LOC = HQ_LOC * DH
WINDOW = 128


def _body(x_ref, wq_ref, kb_ref, vb_ref, wo_ref, out_ref,
          xb_ref, wqi_ref, woi_ref, sc_ref, cwq, cwo, csc, ctx_ref,
          swq, rwq, swo, rwo, ssc, rsc):
    my = lax.axis_index("i")

    wq = wq_ref[...]
    sq = jnp.max(jnp.abs(wq), axis=0, keepdims=True) / 127.0
    sq = jnp.maximum(sq, 1e-8)
    wqi_ref[...] = jnp.round(wq / sq).astype(jnp.int8)
    wo = wo_ref[...]
    so = jnp.max(jnp.abs(wo), axis=1, keepdims=True) / 127.0
    so = jnp.maximum(so, 1e-8)
    woi_ref[...] = jnp.round(wo / so).astype(jnp.int8)
    sc_ref[0:1, :] = sq
    sc_ref[1:2, :] = so.reshape(1, HD_LOC)

    bar = pltpu.get_barrier_semaphore()
    for r in range(1, N_DEV):
        peer = lax.rem(my + r, N_DEV)
        pl.semaphore_signal(bar, inc=1, device_id=(peer,),
                            device_id_type=pl.DeviceIdType.MESH)
    pl.semaphore_wait(bar, N_DEV - 1)

    sends = []
    for r in range(1, N_DEV):
        peer = lax.rem(my + r, N_DEV)
        rd_s = pltpu.make_async_remote_copy(
            src_ref=sc_ref, dst_ref=csc.at[r - 1],
            send_sem=ssc.at[r - 1], recv_sem=rsc.at[r - 1],
            device_id=(peer,), device_id_type=pl.DeviceIdType.MESH)
        rd_q = pltpu.make_async_remote_copy(
            src_ref=wqi_ref, dst_ref=cwq.at[r - 1],
            send_sem=swq.at[r - 1], recv_sem=rwq.at[r - 1],
            device_id=(peer,), device_id_type=pl.DeviceIdType.MESH)
        rd_o = pltpu.make_async_remote_copy(
            src_ref=woi_ref, dst_ref=cwo.at[r - 1],
            send_sem=swo.at[r - 1], recv_sem=rwo.at[r - 1],
            device_id=(peer,), device_id_type=pl.DeviceIdType.MESH)
        rd_s.start()
        rd_q.start()
        rd_o.start()
        sends.append((rd_s, rd_q, rd_o))

    for b in range(B_LOC):
        xb_ref[b * SQ:(b + 1) * SQ, :] = x_ref[b].astype(jnp.bfloat16)

    qi = lax.broadcasted_iota(jnp.int32, (SQ, SKV), 0)
    ki = lax.broadcasted_iota(jnp.int32, (SQ, SKV), 1)
    mask = jnp.abs(qi - ki) <= WINDOW

    def compute_chunk(origin, wq_i8, sq, so, get_wo_i8, first):
        wq_bf = wq_i8.astype(jnp.bfloat16)
        q = jnp.dot(xb_ref[...], wq_bf, preferred_element_type=jnp.float32)
        q = (q * (sq * 0.125)).astype(jnp.bfloat16)
        for b in range(B_LOC):
            kv_start = b * HQ + origin * HQ_LOC
            kb = kb_ref[pl.ds(kv_start, HQ_LOC)]
            vb = vb_ref[pl.ds(kv_start, HQ_LOC)]
            for h in range(HQ_LOC):
                qh = q[b * SQ:(b + 1) * SQ, h * DH:(h + 1) * DH]
                s = lax.dot_general(
                    qh, kb[h], (((1,), (1,)), ((), ())),
                    preferred_element_type=jnp.float32)
                s = jnp.where(mask, s, -1e9)
                s = s - jnp.max(s, axis=1, keepdims=True)
                w = jnp.exp(s)
                w = (w / jnp.sum(w, axis=1, keepdims=True)).astype(jnp.bfloat16)
                ctx = jnp.dot(w, vb[h], preferred_element_type=jnp.float32)
                ctx_ref[b * SQ:(b + 1) * SQ, h * DH:(h + 1) * DH] = (
                    ctx.astype(jnp.bfloat16))
        wo_bf = get_wo_i8().astype(jnp.bfloat16)
        for b in range(B_LOC):
            ctx_s = (ctx_ref[b * SQ:(b + 1) * SQ, :] * so).astype(jnp.bfloat16)
            part = jnp.dot(ctx_s, wo_bf, preferred_element_type=jnp.float32)
            if first:
                out_ref[b] = part
            else:
                out_ref[b] = out_ref[b] + part

    compute_chunk(my, wqi_ref[...], sc_ref[0:1, :], sc_ref[1:2, :],
                  lambda: woi_ref[...], first=True)
    for r in (1, 3, 2):
        rd_s, rd_q, rd_o = sends[r - 1]
        rd_s.wait_recv()
        rd_q.wait_recv()

        def get_wo(r=r, rd_o=rd_o):
            rd_o.wait_recv()
            return cwo[r - 1]

        origin = lax.rem(my - r + N_DEV, N_DEV)
        compute_chunk(origin, cwq[r - 1], csc[r - 1, 0:1, :],
                      csc[r - 1, 1:2, :], get_wo, first=False)

    for rd_s, rd_q, rd_o in sends:
        rd_s.wait_send()
        rd_q.wait_send()
        rd_o.wait_send()


def kernel(x, Wq, K_ext, V_ext, Wo):
    my = lax.axis_index("i")
    Kb = lax.dynamic_slice_in_dim(K_ext, my * B_LOC, B_LOC, axis=0)
    Vb = lax.dynamic_slice_in_dim(V_ext, my * B_LOC, B_LOC, axis=0)
    Kb = Kb.astype(jnp.bfloat16).transpose(0, 2, 1, 3).reshape(
        B_LOC * HQ, SKV, DH)
    Vb = Vb.astype(jnp.bfloat16).transpose(0, 2, 1, 3).reshape(
        B_LOC * HQ, SKV, DH)

    return pl.pallas_call(
        _body,
        out_shape=jax.ShapeDtypeStruct((B_LOC, SQ, D_MODEL), jnp.float32),
        in_specs=[pl.BlockSpec(memory_space=pltpu.VMEM)] * 5,
        out_specs=pl.BlockSpec(memory_space=pltpu.VMEM),
        scratch_shapes=[
            pltpu.VMEM((B_LOC * SQ, D_MODEL), jnp.bfloat16),
            pltpu.VMEM((D_MODEL, HD_LOC), jnp.int8),
            pltpu.VMEM((HD_LOC, D_MODEL), jnp.int8),
            pltpu.VMEM((2, HD_LOC), jnp.float32),
            pltpu.VMEM((N_DEV - 1, D_MODEL, HD_LOC), jnp.int8),
            pltpu.VMEM((N_DEV - 1, HD_LOC, D_MODEL), jnp.int8),
            pltpu.VMEM((N_DEV - 1, 2, HD_LOC), jnp.float32),
            pltpu.VMEM((B_LOC * SQ, HD_LOC), jnp.bfloat16),
            pltpu.SemaphoreType.DMA((N_DEV - 1,)),
            pltpu.SemaphoreType.DMA((N_DEV - 1,)),
            pltpu.SemaphoreType.DMA((N_DEV - 1,)),
            pltpu.SemaphoreType.DMA((N_DEV - 1,)),
            pltpu.SemaphoreType.DMA((N_DEV - 1,)),
            pltpu.SemaphoreType.DMA((N_DEV - 1,)),
        ],
        compiler_params=pltpu.CompilerParams(collective_id=0),
    )(x, Wq, Kb, Vb, Wo)


# device time: 17925 ns/iter; 2.7073x vs baseline; 1.1068x over previous
import jax
import jax.numpy as jnp
from jax import lax
from jax.experimental import pallas as pl
from jax.experimental.pallas import tpu as pltpu

N_DEV = 4
B_LOC = 2
SQ = 256
SKV = 256
HQ = 16
HQ_LOC = 4
DH = 64
D_MODEL = 512
HD_LOC = HQ_LOC * DH
WINDOW = 128


def _body(x_ref, wq_ref, kb_ref, vb_ref, wo_ref, out_ref,
          xb_ref, wqi_ref, woi_ref, sc_ref, cwq, cwo, csc, ctx_ref,
          swq, rwq, swo, rwo, ssc, rsc):
    my = lax.axis_index("i")

    wq = wq_ref[...]
    sq = jnp.max(jnp.abs(wq), axis=0, keepdims=True) / 127.0
    sq = jnp.maximum(sq, 1e-8)
    wqi_ref[...] = jnp.round(wq / sq).astype(jnp.int8)
    wo = wo_ref[...]
    so = jnp.max(jnp.abs(wo), axis=1, keepdims=True) / 127.0
    so = jnp.maximum(so, 1e-8)
    woi_ref[...] = jnp.round(wo / so).astype(jnp.int8)
    sc_ref[0:1, :] = sq
    sc_ref[1:2, :] = so.reshape(1, HD_LOC)

    bar = pltpu.get_barrier_semaphore()
    for r in range(1, N_DEV):
        peer = lax.rem(my + r, N_DEV)
        pl.semaphore_signal(bar, inc=1, device_id=(peer,),
                            device_id_type=pl.DeviceIdType.MESH)
    pl.semaphore_wait(bar, N_DEV - 1)

    sends = []
    for r in range(1, N_DEV):
        peer = lax.rem(my + r, N_DEV)
        rd_s = pltpu.make_async_remote_copy(
            src_ref=sc_ref, dst_ref=csc.at[r - 1],
            send_sem=ssc.at[r - 1], recv_sem=rsc.at[r - 1],
            device_id=(peer,), device_id_type=pl.DeviceIdType.MESH)
        rd_q = pltpu.make_async_remote_copy(
            src_ref=wqi_ref, dst_ref=cwq.at[r - 1],
            send_sem=swq.at[r - 1], recv_sem=rwq.at[r - 1],
            device_id=(peer,), device_id_type=pl.DeviceIdType.MESH)
        rd_o = pltpu.make_async_remote_copy(
            src_ref=woi_ref, dst_ref=cwo.at[r - 1],
            send_sem=swo.at[r - 1], recv_sem=rwo.at[r - 1],
            device_id=(peer,), device_id_type=pl.DeviceIdType.MESH)
        rd_s.start()
        rd_q.start()
        rd_o.start()
        sends.append((rd_s, rd_q, rd_o))

    for b in range(B_LOC):
        xb_ref[b * SQ:(b + 1) * SQ, :] = x_ref[b].astype(jnp.bfloat16)

    qi = lax.broadcasted_iota(jnp.int32, (SQ, SKV), 0)
    ki = lax.broadcasted_iota(jnp.int32, (SQ, SKV), 1)
    mask = jnp.abs(qi - ki) <= WINDOW

    def compute_chunk(origin, wq_i8, sq, so, get_wo_i8, first):
        wq_bf = wq_i8.astype(jnp.bfloat16)
        q = jnp.dot(xb_ref[...], wq_bf, preferred_element_type=jnp.float32)
        q = (q * (sq * 0.125)).astype(jnp.bfloat16)
        for b in range(B_LOC):
            kv_start = b * HQ + origin * HQ_LOC
            kb = kb_ref[pl.ds(kv_start, HQ_LOC)]
            vb = vb_ref[pl.ds(kv_start, HQ_LOC)]
            for h in range(HQ_LOC):
                qh = q[b * SQ:(b + 1) * SQ, h * DH:(h + 1) * DH]
                s = lax.dot_general(
                    qh, kb[h], (((1,), (1,)), ((), ())),
                    preferred_element_type=jnp.float32)
                e = jnp.where(mask, jnp.exp(s), 0.0)
                denom = jnp.sum(e, axis=1, keepdims=True)
                ctx = jnp.dot(e.astype(jnp.bfloat16), vb[h],
                              preferred_element_type=jnp.float32) / denom
                ctx_ref[b * SQ:(b + 1) * SQ, h * DH:(h + 1) * DH] = (
                    ctx.astype(jnp.bfloat16))
        wo_bf = get_wo_i8().astype(jnp.bfloat16)
        for b in range(B_LOC):
            ctx_s = (ctx_ref[b * SQ:(b + 1) * SQ, :] * so).astype(jnp.bfloat16)
            part = jnp.dot(ctx_s, wo_bf, preferred_element_type=jnp.float32)
            if first:
                out_ref[b] = part
            else:
                out_ref[b] = out_ref[b] + part

    compute_chunk(my, wqi_ref[...], sc_ref[0:1, :], sc_ref[1:2, :],
                  lambda: woi_ref[...], first=True)
    for r in (1, 3, 2):
        rd_s, rd_q, rd_o = sends[r - 1]
        rd_s.wait_recv()
        rd_q.wait_recv()

        def get_wo(r=r, rd_o=rd_o):
            rd_o.wait_recv()
            return cwo[r - 1]

        origin = lax.rem(my - r + N_DEV, N_DEV)
        compute_chunk(origin, cwq[r - 1], csc[r - 1, 0:1, :],
                      csc[r - 1, 1:2, :], get_wo, first=False)

    for rd_s, rd_q, rd_o in sends:
        rd_s.wait_send()
        rd_q.wait_send()
        rd_o.wait_send()


def kernel(x, Wq, K_ext, V_ext, Wo):
    my = lax.axis_index("i")
    Kb = lax.dynamic_slice_in_dim(K_ext, my * B_LOC, B_LOC, axis=0)
    Vb = lax.dynamic_slice_in_dim(V_ext, my * B_LOC, B_LOC, axis=0)
    Kb = Kb.astype(jnp.bfloat16).transpose(0, 2, 1, 3).reshape(
        B_LOC * HQ, SKV, DH)
    Vb = Vb.astype(jnp.bfloat16).transpose(0, 2, 1, 3).reshape(
        B_LOC * HQ, SKV, DH)

    return pl.pallas_call(
        _body,
        out_shape=jax.ShapeDtypeStruct((B_LOC, SQ, D_MODEL), jnp.float32),
        in_specs=[pl.BlockSpec(memory_space=pltpu.VMEM)] * 5,
        out_specs=pl.BlockSpec(memory_space=pltpu.VMEM),
        scratch_shapes=[
            pltpu.VMEM((B_LOC * SQ, D_MODEL), jnp.bfloat16),
            pltpu.VMEM((D_MODEL, HD_LOC), jnp.int8),
            pltpu.VMEM((HD_LOC, D_MODEL), jnp.int8),
            pltpu.VMEM((2, HD_LOC), jnp.float32),
            pltpu.VMEM((N_DEV - 1, D_MODEL, HD_LOC), jnp.int8),
            pltpu.VMEM((N_DEV - 1, HD_LOC, D_MODEL), jnp.int8),
            pltpu.VMEM((N_DEV - 1, 2, HD_LOC), jnp.float32),
            pltpu.VMEM((B_LOC * SQ, HD_LOC), jnp.bfloat16),
            pltpu.SemaphoreType.DMA((N_DEV - 1,)),
            pltpu.SemaphoreType.DMA((N_DEV - 1,)),
            pltpu.SemaphoreType.DMA((N_DEV - 1,)),
            pltpu.SemaphoreType.DMA((N_DEV - 1,)),
            pltpu.SemaphoreType.DMA((N_DEV - 1,)),
            pltpu.SemaphoreType.DMA((N_DEV - 1,)),
        ],
        compiler_params=pltpu.CompilerParams(collective_id=0),
    )(x, Wq, Kb, Vb, Wo)
